# baseline (speedup 1.0000x reference)
import functools

import jax
import jax.numpy as jnp
from jax.experimental import pallas as pl
from jax.experimental.pallas import tpu as pltpu

_DNT = (((1,), (1,)), ((), ()))

_IN_SLOTS = 4
_OUT_SLOTS = 6


def _manual_kernel(half, st, x_hbm, w1_ref, b1_ref, w2_ref, b2_ref,
                   fr_hbm, x0t_hbm, xbuf, frbuf, x0tbuf, insem, outsem, x0sem):
    core = pl.program_id(0)
    base = core * half
    n_steps = half // st
    for s in range(n_steps):
        islot = s % _IN_SLOTS
        oslot = s % _OUT_SLOTS
        if s == 0:
            for p in range(min(_IN_SLOTS, n_steps)):
                pltpu.make_async_copy(
                    x_hbm.at[pl.ds(base + p * st, st), :],
                    xbuf.at[p], insem.at[p]).start()
        elif s + _IN_SLOTS - 1 < n_steps:
            nxt = s + _IN_SLOTS - 1
            pltpu.make_async_copy(
                x_hbm.at[pl.ds(base + nxt * st, st), :],
                xbuf.at[nxt % _IN_SLOTS], insem.at[nxt % _IN_SLOTS]).start()
        pltpu.make_async_copy(xbuf.at[islot], xbuf.at[islot],
                              insem.at[islot]).wait()
        if s >= _OUT_SLOTS:
            pltpu.make_async_copy(frbuf.at[oslot], frbuf.at[oslot],
                                  outsem.at[oslot]).wait()
            pltpu.make_async_copy(x0tbuf.at[oslot], x0tbuf.at[oslot],
                                  x0sem.at[oslot]).wait()
        x = xbuf[islot]
        x0 = jax.lax.dot_general(x, w1_ref[...], _DNT,
                                 preferred_element_type=jnp.float32)
        x0 = x0 + b1_ref[...]
        x0tbuf[oslot] = x0.T
        z = jax.lax.dot_general(x0, w2_ref[...], _DNT,
                                preferred_element_type=jnp.float32)
        frbuf[oslot] = jnp.exp(z + b2_ref[...])
        pltpu.make_async_copy(
            frbuf.at[oslot], fr_hbm.at[pl.ds(base + s * st, st), :],
            outsem.at[oslot]).start()
        pltpu.make_async_copy(
            x0tbuf.at[oslot], x0t_hbm.at[:, pl.ds(base + s * st, st)],
            x0sem.at[oslot]).start()
    for p in range(min(_OUT_SLOTS, n_steps)):
        pltpu.make_async_copy(frbuf.at[p], frbuf.at[p], outsem.at[p]).wait()
        pltpu.make_async_copy(x0tbuf.at[p], x0tbuf.at[p], x0sem.at[p]).wait()


@functools.partial(jax.jit, static_argnames=("subtile",))
def _lnp_manual(x, w1, b1, w2, b2, *, subtile=2048):
    B, D = x.shape
    H = w1.shape[0]
    N = w2.shape[0]
    half = B // 2
    st = subtile

    fr, x0t = pl.pallas_call(
        functools.partial(_manual_kernel, half, st),
        out_shape=(
            jax.ShapeDtypeStruct((B, N), jnp.float32),
            jax.ShapeDtypeStruct((H, B), jnp.float32),
        ),
        grid=(2,),
        in_specs=[
            pl.BlockSpec(memory_space=pl.ANY),
            pl.BlockSpec((H, D), lambda i: (0, 0)),
            pl.BlockSpec((1, H), lambda i: (0, 0)),
            pl.BlockSpec((N, H), lambda i: (0, 0)),
            pl.BlockSpec((1, N), lambda i: (0, 0)),
        ],
        out_specs=(
            pl.BlockSpec(memory_space=pl.ANY),
            pl.BlockSpec(memory_space=pl.ANY),
        ),
        scratch_shapes=[
            pltpu.VMEM((_IN_SLOTS, st, D), jnp.float32),
            pltpu.VMEM((_OUT_SLOTS, st, N), jnp.float32),
            pltpu.VMEM((_OUT_SLOTS, H, st), jnp.float32),
            pltpu.SemaphoreType.DMA((_IN_SLOTS,)),
            pltpu.SemaphoreType.DMA((_OUT_SLOTS,)),
            pltpu.SemaphoreType.DMA((_OUT_SLOTS,)),
        ],
        compiler_params=pltpu.CompilerParams(
            dimension_semantics=("parallel",),
        ),
    )(x, w1, b1.reshape(1, H), w2, b2.reshape(1, N))

    return fr, x0t[:, :B].T


def kernel(x, w1, b1, w2, b2):
    return _lnp_manual(x, w1, b1, w2, b2, subtile=1024)


# speedup vs baseline: 1.0349x; 1.0349x over previous
import functools

import jax
import jax.numpy as jnp
from jax.experimental import pallas as pl
from jax.experimental.pallas import tpu as pltpu

_DNT = (((1,), (1,)), ((), ()))


def _round_up(v, m):
    return ((v + m - 1) // m) * m


def _mlp_exp_kernel(x_ref, w1_ref, b1_ref, w2_ref, b2_ref, fr_ref, x0t_ref):
    x = x_ref[...]
    x0 = jax.lax.dot_general(x, w1_ref[...], _DNT,
                             preferred_element_type=jnp.float32)
    x0 = x0 + b1_ref[...]
    x0t_ref[...] = x0.T
    n = fr_ref.shape[-1]
    nc = 256
    for j in range(0, n, nc):
        z = jax.lax.dot_general(x0, w2_ref[j:j + nc, :], _DNT,
                                preferred_element_type=jnp.float32)
        fr_ref[:, j:j + nc] = jnp.exp(z + b2_ref[:, j:j + nc])


@functools.partial(jax.jit, static_argnames=("block_b",))
def _lnp_forward(x, w1, b1, w2, b2, *, block_b=4096):
    B, D = x.shape
    H = w1.shape[0]
    N = w2.shape[0]

    TB = min(block_b, _round_up(B, 8))
    Bp = _round_up(B, TB)
    x_in = x
    if Bp != B:
        x_in = jnp.zeros((Bp, D), x.dtype).at[:B, :].set(x)

    fr, x0t = pl.pallas_call(
        _mlp_exp_kernel,
        out_shape=(
            jax.ShapeDtypeStruct((Bp, N), jnp.float32),
            jax.ShapeDtypeStruct((H, Bp), jnp.float32),
        ),
        grid=(Bp // TB,),
        in_specs=[
            pl.BlockSpec((TB, D), lambda i: (i, 0)),
            pl.BlockSpec((H, D), lambda i: (0, 0)),
            pl.BlockSpec((1, H), lambda i: (0, 0)),
            pl.BlockSpec((N, H), lambda i: (0, 0)),
            pl.BlockSpec((1, N), lambda i: (0, 0)),
        ],
        out_specs=(
            pl.BlockSpec((TB, N), lambda i: (i, 0)),
            pl.BlockSpec((H, TB), lambda i: (0, i)),
        ),
        compiler_params=pltpu.CompilerParams(
            dimension_semantics=("parallel",),
        ),
    )(x_in, w1, b1.reshape(1, H), w2, b2.reshape(1, N))

    return fr[:B], x0t[:, :B].T


def kernel(x, w1, b1, w2, b2):
    return _lnp_forward(x, w1, b1, w2, b2, block_b=8192)
